# baseline (speedup 1.0000x reference)
import functools

import jax
import jax.numpy as jnp
from jax import lax
from jax.experimental import pallas as pl
from jax.experimental.pallas import tpu as pltpu
from jax.experimental.pallas import tpu_sc as plsc

B = 16384
V = 100000
D = 300
E = 128
CHUNK = 128
TAIL = D - 256


def _tc_tailpack(attr_table, obj_table):
    R = 2000
    NB = V // R

    def body(a_ref, o_ref, t_ref):
        lane = lax.broadcasted_iota(jnp.int32, (R, E), 1)
        a = jnp.where(lane < TAIL, a_ref[...], 0.0)
        o = jnp.where(lane < TAIL, o_ref[...], 0.0)
        t_ref[...] = a + jnp.concatenate(
            [jnp.zeros((R, TAIL), jnp.float32), o[:, :E - TAIL]], axis=1)

    return pl.pallas_call(
        body,
        grid=(NB,),
        in_specs=[
            pl.BlockSpec((R, E), lambda i: (i, 2)),
            pl.BlockSpec((R, E), lambda i: (i, 2)),
        ],
        out_specs=pl.BlockSpec((R, E), lambda i: (i, 0)),
        out_shape=jax.ShapeDtypeStruct((V, E), jnp.float32),
    )(attr_table, obj_table)


def _sc_main(attrs2d, objs2d, attr_table, obj_table, nc, ns):
    NW = nc * ns
    C = B // NW // CHUNK
    mesh = plsc.VectorSubcoreMesh(core_axis_name="c", subcore_axis_name="s")

    @functools.partial(
        pl.kernel,
        out_type=tuple(
            jax.ShapeDtypeStruct((B, 128), jnp.float32) for _ in range(4)),
        mesh=mesh,
        scratch_types=[
            pltpu.VMEM((C, CHUNK), jnp.int32),
            pltpu.VMEM((C, CHUNK), jnp.int32),
            pltpu.VMEM((CHUNK, 128), jnp.float32),
            pltpu.VMEM((CHUNK, 128), jnp.float32),
            pltpu.VMEM((CHUNK, 128), jnp.float32),
            pltpu.VMEM((CHUNK, 128), jnp.float32),
            pltpu.SemaphoreType.DMA,
        ],
    )
    def k(attrs_h, objs_h, atab_h, otab_h, a0_h, a1_h, o0_h, o1_h, ia, io,
          bufa0, bufa1, bufb0, bufb1, sem):
        wid = lax.axis_index("s") * nc + lax.axis_index("c")
        base = wid * (C * CHUNK)
        pltpu.sync_copy(attrs_h.at[wid], ia)
        pltpu.sync_copy(objs_h.at[wid], io)
        bufs0 = (bufa0, bufa1)
        bufs1 = (bufb0, bufb1)
        jobs = [(ia, c, atab_h, a0_h, a1_h) for c in range(C)]
        jobs += [(io, c, otab_h, o0_h, o1_h) for c in range(C)]

        def fire(j):
            idxr, c, tab, _, _ = jobs[j]
            h0 = pltpu.async_copy(tab.at[idxr.at[c], pl.ds(0, 128)],
                                  bufs0[j % 2], sem)
            h1 = pltpu.async_copy(tab.at[idxr.at[c], pl.ds(128, 128)],
                                  bufs1[j % 2], sem)
            return h0, h1

        hs = fire(0)
        for j in range(len(jobs)):
            hs[0].wait()
            hs[1].wait()
            if j + 1 < len(jobs):
                hs = fire(j + 1)
            _, c, _, out0, out1 = jobs[j]
            rows = pl.ds(base + c * CHUNK, CHUNK)
            pltpu.sync_copy(bufs0[j % 2], out0.at[rows])
            pltpu.sync_copy(bufs1[j % 2], out1.at[rows])

    return k(attrs2d, objs2d, attr_table, obj_table)


def _sc_tail(attrs2d, objs2d, tailpack, nc, ns):
    NW = nc * ns
    C = B // NW // CHUNK
    mesh = plsc.VectorSubcoreMesh(core_axis_name="c", subcore_axis_name="s")

    @functools.partial(
        pl.kernel,
        out_type=tuple(
            jax.ShapeDtypeStruct((B, 128), jnp.float32) for _ in range(2)),
        mesh=mesh,
        scratch_types=[
            pltpu.VMEM((C, CHUNK), jnp.int32),
            pltpu.VMEM((C, CHUNK), jnp.int32),
            pltpu.VMEM((CHUNK, 128), jnp.float32),
            pltpu.VMEM((CHUNK, 128), jnp.float32),
            pltpu.SemaphoreType.DMA,
        ],
    )
    def k(attrs_h, objs_h, tp_h, at_h, ot_h, ia, io, buf0, buf1, sem):
        wid = lax.axis_index("s") * nc + lax.axis_index("c")
        base = wid * (C * CHUNK)
        pltpu.sync_copy(attrs_h.at[wid], ia)
        pltpu.sync_copy(objs_h.at[wid], io)
        bufs = (buf0, buf1)
        jobs = [(ia, c, at_h) for c in range(C)]
        jobs += [(io, c, ot_h) for c in range(C)]

        def fire(j):
            idxr, c, _ = jobs[j]
            return pltpu.async_copy(tp_h.at[idxr.at[c]], bufs[j % 2], sem)

        h = fire(0)
        for j in range(len(jobs)):
            h.wait()
            if j + 1 < len(jobs):
                h = fire(j + 1)
            _, c, out = jobs[j]
            pltpu.sync_copy(bufs[j % 2], out.at[pl.ds(base + c * CHUNK,
                                                      CHUNK)])

    return k(attrs2d, objs2d, tailpack)


def _tc_project(a0, a1, at, o0, o1, ot, Wa, ba, Wo, bo):
    NB = 32
    R = B // NB

    def body(a0_ref, a1_ref, at_ref, o0_ref, o1_ref, ot_ref, wa_ref, ba_ref,
             wo_ref, bo_ref, oa_ref, oo_ref):
        def leaky(x):
            return jnp.where(x >= 0, x, 0.01 * x)

        def one(p0, p1, pt, w_ref, b_ref, o_ref):
            y = jnp.dot(leaky(p0[...]), w_ref[0:128],
                        preferred_element_type=jnp.float32)
            y += jnp.dot(leaky(p1[...]), w_ref[128:256],
                         preferred_element_type=jnp.float32)
            y += jnp.dot(leaky(pt[...]), w_ref[256:384],
                         preferred_element_type=jnp.float32)
            y += b_ref[...][None, :]
            n = jnp.sqrt(jnp.sum(y * y, axis=1, keepdims=True))
            o_ref[...] = y / jnp.maximum(n, 1e-12)

        one(a0_ref, a1_ref, at_ref, wa_ref, ba_ref, oa_ref)
        one(o0_ref, o1_ref, ot_ref, wo_ref, bo_ref, oo_ref)

    blk = lambda: pl.BlockSpec((R, 128), lambda i: (i, 0))
    wblk = lambda: pl.BlockSpec((384, E), lambda i: (0, 0))
    bblk = lambda: pl.BlockSpec((E,), lambda i: (0,))
    return pl.pallas_call(
        body,
        grid=(NB,),
        in_specs=[blk(), blk(), blk(), blk(), blk(), blk(),
                  wblk(), bblk(), wblk(), bblk()],
        out_specs=[
            pl.BlockSpec((R, E), lambda i: (i, 0)),
            pl.BlockSpec((R, E), lambda i: (i, 0)),
        ],
        out_shape=[
            jax.ShapeDtypeStruct((B, E), jnp.float32),
            jax.ShapeDtypeStruct((B, E), jnp.float32),
        ],
    )(a0, a1, at, o0, o1, ot, Wa, ba, Wo, bo)


def kernel(attrs, objs, attr_table, obj_table, W_attr, b_attr, W_obj, b_obj):
    info = plsc.get_sparse_core_info()
    nc, ns = info.num_cores, info.num_subcores
    NW = nc * ns
    C = B // NW // CHUNK
    a2 = attrs.astype(jnp.int32).reshape(NW, C, CHUNK)
    o2 = objs.astype(jnp.int32).reshape(NW, C, CHUNK)

    zeros = jnp.zeros((E - 2 * TAIL, E), jnp.float32)
    Wa_eff = jnp.concatenate(
        [W_attr[:256], W_attr[256:], jnp.zeros((E - TAIL, E), jnp.float32)],
        axis=0)
    Wo_eff = jnp.concatenate(
        [W_obj[:256], jnp.zeros((TAIL, E), jnp.float32), W_obj[256:], zeros],
        axis=0)

    a0, a1, o0, o1 = _sc_main(a2, o2, attr_table, obj_table, nc, ns)
    tp = jnp.concatenate(
        [attr_table[:, 256:], obj_table[:, 256:],
         jnp.zeros((V, E - 2 * TAIL), jnp.float32)], axis=1)
    at, ot = _sc_tail(a2, o2, tp, nc, ns)
    oa, oo = _tc_project(a0, a1, at, o0, o1, ot, Wa_eff, b_attr, Wo_eff,
                         b_obj)
    return oa, oo


# speedup vs baseline: 1.1572x; 1.1572x over previous
import functools

import jax
import jax.numpy as jnp
from jax import lax
from jax.experimental import pallas as pl
from jax.experimental.pallas import tpu as pltpu
from jax.experimental.pallas import tpu_sc as plsc

B = 16384
V = 100000
D = 300
E = 128
CHUNK = 128
TAIL = D - 256


def _tc_tailpack(attr_table, obj_table):
    R = 10000
    NB = V // R

    def body(a_ref, o_ref, t_ref):
        lane = lax.broadcasted_iota(jnp.int32, (R, E), 1)
        a = jnp.where(lane < TAIL, a_ref[...], 0.0)
        o = jnp.where(lane < TAIL, o_ref[...], 0.0)
        t_ref[...] = a + jnp.concatenate(
            [jnp.zeros((R, TAIL), jnp.float32), o[:, :E - TAIL]], axis=1)

    return pl.pallas_call(
        body,
        grid=(NB,),
        in_specs=[
            pl.BlockSpec((R, E), lambda i: (i, 2)),
            pl.BlockSpec((R, E), lambda i: (i, 2)),
        ],
        out_specs=pl.BlockSpec((R, E), lambda i: (i, 0)),
        out_shape=jax.ShapeDtypeStruct((V, E), jnp.float32),
    )(attr_table, obj_table)


def _sc_main(attrs2d, objs2d, attr_table, obj_table, nc, ns):
    NW = nc * ns
    C = B // NW // CHUNK
    mesh = plsc.VectorSubcoreMesh(core_axis_name="c", subcore_axis_name="s")

    @functools.partial(
        pl.kernel,
        out_type=tuple(
            jax.ShapeDtypeStruct((B, 128), jnp.float32) for _ in range(4)),
        mesh=mesh,
        scratch_types=[
            pltpu.VMEM((C, CHUNK), jnp.int32),
            pltpu.VMEM((C, CHUNK), jnp.int32),
            pltpu.VMEM((CHUNK, 128), jnp.float32),
            pltpu.VMEM((CHUNK, 128), jnp.float32),
            pltpu.VMEM((CHUNK, 128), jnp.float32),
            pltpu.VMEM((CHUNK, 128), jnp.float32),
            pltpu.SemaphoreType.DMA,
        ],
    )
    def k(attrs_h, objs_h, atab_h, otab_h, a0_h, a1_h, o0_h, o1_h, ia, io,
          bufa0, bufa1, bufb0, bufb1, sem):
        wid = lax.axis_index("s") * nc + lax.axis_index("c")
        base = wid * (C * CHUNK)
        pltpu.sync_copy(attrs_h.at[wid], ia)
        pltpu.sync_copy(objs_h.at[wid], io)
        bufs0 = (bufa0, bufa1)
        bufs1 = (bufb0, bufb1)
        jobs = [(ia, c, atab_h, a0_h, a1_h) for c in range(C)]
        jobs += [(io, c, otab_h, o0_h, o1_h) for c in range(C)]

        def fire(j):
            idxr, c, tab, _, _ = jobs[j]
            h0 = pltpu.async_copy(tab.at[idxr.at[c], pl.ds(0, 128)],
                                  bufs0[j % 2], sem)
            h1 = pltpu.async_copy(tab.at[idxr.at[c], pl.ds(128, 128)],
                                  bufs1[j % 2], sem)
            return h0, h1

        hs = fire(0)
        for j in range(len(jobs)):
            hs[0].wait()
            hs[1].wait()
            if j + 1 < len(jobs):
                hs = fire(j + 1)
            _, c, _, out0, out1 = jobs[j]
            rows = pl.ds(base + c * CHUNK, CHUNK)
            pltpu.sync_copy(bufs0[j % 2], out0.at[rows])
            pltpu.sync_copy(bufs1[j % 2], out1.at[rows])

    return k(attrs2d, objs2d, attr_table, obj_table)


def _sc_tail(attrs2d, objs2d, tailpack, nc, ns):
    NW = nc * ns
    C = B // NW // CHUNK
    mesh = plsc.VectorSubcoreMesh(core_axis_name="c", subcore_axis_name="s")

    @functools.partial(
        pl.kernel,
        out_type=tuple(
            jax.ShapeDtypeStruct((B, 128), jnp.float32) for _ in range(2)),
        mesh=mesh,
        scratch_types=[
            pltpu.VMEM((C, CHUNK), jnp.int32),
            pltpu.VMEM((C, CHUNK), jnp.int32),
            pltpu.VMEM((CHUNK, 128), jnp.float32),
            pltpu.VMEM((CHUNK, 128), jnp.float32),
            pltpu.SemaphoreType.DMA,
        ],
    )
    def k(attrs_h, objs_h, tp_h, at_h, ot_h, ia, io, buf0, buf1, sem):
        wid = lax.axis_index("s") * nc + lax.axis_index("c")
        base = wid * (C * CHUNK)
        pltpu.sync_copy(attrs_h.at[wid], ia)
        pltpu.sync_copy(objs_h.at[wid], io)
        bufs = (buf0, buf1)
        jobs = [(ia, c, at_h) for c in range(C)]
        jobs += [(io, c, ot_h) for c in range(C)]

        def fire(j):
            idxr, c, _ = jobs[j]
            return pltpu.async_copy(tp_h.at[idxr.at[c]], bufs[j % 2], sem)

        h = fire(0)
        for j in range(len(jobs)):
            h.wait()
            if j + 1 < len(jobs):
                h = fire(j + 1)
            _, c, out = jobs[j]
            pltpu.sync_copy(bufs[j % 2], out.at[pl.ds(base + c * CHUNK,
                                                      CHUNK)])

    return k(attrs2d, objs2d, tailpack)


def _tc_project(a0, a1, at, o0, o1, ot, Wa, ba, Wo, bo):
    NB = 32
    R = B // NB

    def body(a0_ref, a1_ref, at_ref, o0_ref, o1_ref, ot_ref, wa_ref, ba_ref,
             wo_ref, bo_ref, oa_ref, oo_ref):
        def leaky(x):
            return jnp.where(x >= 0, x, 0.01 * x)

        def one(p0, p1, pt, w_ref, b_ref, o_ref):
            y = jnp.dot(leaky(p0[...]), w_ref[0:128],
                        preferred_element_type=jnp.float32)
            y += jnp.dot(leaky(p1[...]), w_ref[128:256],
                         preferred_element_type=jnp.float32)
            y += jnp.dot(leaky(pt[...]), w_ref[256:384],
                         preferred_element_type=jnp.float32)
            y += b_ref[...][None, :]
            n = jnp.sqrt(jnp.sum(y * y, axis=1, keepdims=True))
            o_ref[...] = y / jnp.maximum(n, 1e-12)

        one(a0_ref, a1_ref, at_ref, wa_ref, ba_ref, oa_ref)
        one(o0_ref, o1_ref, ot_ref, wo_ref, bo_ref, oo_ref)

    blk = lambda: pl.BlockSpec((R, 128), lambda i: (i, 0))
    wblk = lambda: pl.BlockSpec((384, E), lambda i: (0, 0))
    bblk = lambda: pl.BlockSpec((E,), lambda i: (0,))
    return pl.pallas_call(
        body,
        grid=(NB,),
        in_specs=[blk(), blk(), blk(), blk(), blk(), blk(),
                  wblk(), bblk(), wblk(), bblk()],
        out_specs=[
            pl.BlockSpec((R, E), lambda i: (i, 0)),
            pl.BlockSpec((R, E), lambda i: (i, 0)),
        ],
        out_shape=[
            jax.ShapeDtypeStruct((B, E), jnp.float32),
            jax.ShapeDtypeStruct((B, E), jnp.float32),
        ],
    )(a0, a1, at, o0, o1, ot, Wa, ba, Wo, bo)


def kernel(attrs, objs, attr_table, obj_table, W_attr, b_attr, W_obj, b_obj):
    info = plsc.get_sparse_core_info()
    nc, ns = info.num_cores, info.num_subcores
    NW = nc * ns
    C = B // NW // CHUNK
    a2 = attrs.astype(jnp.int32).reshape(NW, C, CHUNK)
    o2 = objs.astype(jnp.int32).reshape(NW, C, CHUNK)

    zeros = jnp.zeros((E - 2 * TAIL, E), jnp.float32)
    Wa_eff = jnp.concatenate(
        [W_attr[:256], W_attr[256:], jnp.zeros((E - TAIL, E), jnp.float32)],
        axis=0)
    Wo_eff = jnp.concatenate(
        [W_obj[:256], jnp.zeros((TAIL, E), jnp.float32), W_obj[256:], zeros],
        axis=0)

    a0, a1, o0, o1 = _sc_main(a2, o2, attr_table, obj_table, nc, ns)
    tp = _tc_tailpack(attr_table, obj_table)
    at, ot = _sc_tail(a2, o2, tp, nc, ns)
    oa, oo = _tc_project(a0, a1, at, o0, o1, ot, Wa_eff, b_attr, Wo_eff,
                         b_obj)
    return oa, oo


# speedup vs baseline: 2.3165x; 2.0019x over previous
import functools

import jax
import jax.numpy as jnp
from jax import lax
from jax.experimental import pallas as pl
from jax.experimental.pallas import tpu as pltpu
from jax.experimental.pallas import tpu_sc as plsc

B = 16384
V = 100000
D = 300
E = 128
CHUNK = 128
NC = 2048


def _tc_preproject(tabT, W, b):
    grid = (pl.cdiv(V, NC),)

    def body(x_ref, w_ref, b_ref, z_ref):
        x = x_ref[...]
        x = jnp.where(x >= 0, x, 0.01 * x)
        y = lax.dot_general(x, w_ref[...], (((0,), (0,)), ((), ())),
                            preferred_element_type=jnp.float32)
        y = y + b_ref[...][None, :]
        n = jnp.sqrt(jnp.sum(y * y, axis=1, keepdims=True))
        z_ref[...] = y / jnp.maximum(n, 1e-12)

    return pl.pallas_call(
        body,
        grid=grid,
        in_specs=[
            pl.BlockSpec((D, NC), lambda i: (0, i)),
            pl.BlockSpec((D, E), lambda i: (0, 0)),
            pl.BlockSpec((E,), lambda i: (0,)),
        ],
        out_specs=pl.BlockSpec((NC, E), lambda i: (i, 0)),
        out_shape=jax.ShapeDtypeStruct((V, E), jnp.float32),
    )(tabT, W, b)


def _sc_gather(idx2d, z, nc, ns):
    NW = nc * ns
    C = B // NW // CHUNK
    mesh = plsc.VectorSubcoreMesh(core_axis_name="c", subcore_axis_name="s")

    @functools.partial(
        pl.kernel,
        out_type=(jax.ShapeDtypeStruct((B, E), jnp.float32),),
        mesh=mesh,
        scratch_types=[
            pltpu.VMEM((C, CHUNK), jnp.int32),
            pltpu.VMEM((CHUNK, E), jnp.float32),
            pltpu.VMEM((CHUNK, E), jnp.float32),
            pltpu.SemaphoreType.DMA,
        ],
    )
    def k(idx_h, z_h, out_h, iv, buf0, buf1, sem):
        wid = lax.axis_index("s") * nc + lax.axis_index("c")
        base = wid * (C * CHUNK)
        pltpu.sync_copy(idx_h.at[wid], iv)
        bufs = (buf0, buf1)

        def fire(c):
            return pltpu.async_copy(z_h.at[iv.at[c]], bufs[c % 2], sem)

        h = fire(0)
        for c in range(C):
            h.wait()
            if c + 1 < C:
                h = fire(c + 1)
            pltpu.sync_copy(bufs[c % 2], out_h.at[pl.ds(base + c * CHUNK,
                                                        CHUNK)])

    return k(idx2d, z)[0]


def kernel(attrs, objs, attr_table, obj_table, W_attr, b_attr, W_obj, b_obj):
    info = plsc.get_sparse_core_info()
    nc, ns = info.num_cores, info.num_subcores
    NW = nc * ns
    C = B // NW // CHUNK
    a2 = attrs.astype(jnp.int32).reshape(NW, C, CHUNK)
    o2 = objs.astype(jnp.int32).reshape(NW, C, CHUNK)

    za = _tc_preproject(attr_table.T, W_attr, b_attr)
    oa = _sc_gather(a2, za, nc, ns)
    zo = _tc_preproject(obj_table.T, W_obj, b_obj)
    oo = _sc_gather(o2, zo, nc, ns)
    return oa, oo


# speedup vs baseline: 2.4727x; 1.0674x over previous
import functools

import jax
import jax.numpy as jnp
from jax import lax
from jax.experimental import pallas as pl
from jax.experimental.pallas import tpu as pltpu
from jax.experimental.pallas import tpu_sc as plsc

B = 16384
V = 100000
D = 300
E = 128
CHUNK = 128
NC = 2048


def _tc_preproject_pack(tabTa, tabTo, Wa, ba, Wo, bo):
    grid = (pl.cdiv(V, NC),)

    def one(x, w_ref, b_ref):
        x = jnp.where(x >= 0, x, 0.01 * x)
        y = lax.dot_general(x, w_ref[...], (((0,), (0,)), ((), ())),
                            preferred_element_type=jnp.float32)
        y = y + b_ref[...][None, :]
        n = jnp.sqrt(jnp.sum(y * y, axis=1, keepdims=True))
        return y / jnp.maximum(n, 1e-12)

    def body(xa_ref, xo_ref, wa_ref, ba_ref, wo_ref, bo_ref, z_ref):
        za = one(xa_ref[...], wa_ref, ba_ref).astype(jnp.bfloat16)
        zo = one(xo_ref[...], wo_ref, bo_ref).astype(jnp.bfloat16)
        ua = lax.bitcast_convert_type(za, jnp.uint16).astype(jnp.uint32)
        uo = lax.bitcast_convert_type(zo, jnp.uint16).astype(jnp.uint32)
        packed = ua | (uo << 16)
        z_ref[...] = lax.bitcast_convert_type(packed, jnp.float32)

    return pl.pallas_call(
        body,
        grid=grid,
        in_specs=[
            pl.BlockSpec((D, NC), lambda i: (0, i)),
            pl.BlockSpec((D, NC), lambda i: (0, i)),
            pl.BlockSpec((D, E), lambda i: (0, 0)),
            pl.BlockSpec((E,), lambda i: (0,)),
            pl.BlockSpec((D, E), lambda i: (0, 0)),
            pl.BlockSpec((E,), lambda i: (0,)),
        ],
        out_specs=pl.BlockSpec((NC, E), lambda i: (i, 0)),
        out_shape=jax.ShapeDtypeStruct((V, E), jnp.float32),
    )(tabTa, tabTo, Wa, ba, Wo, bo)


def _sc_gather(idx2d, z, nc, ns):
    NW = nc * ns
    C = B // NW // CHUNK
    mesh = plsc.VectorSubcoreMesh(core_axis_name="c", subcore_axis_name="s")

    @functools.partial(
        pl.kernel,
        out_type=(jax.ShapeDtypeStruct((B, E), jnp.float32),),
        mesh=mesh,
        scratch_types=[
            pltpu.VMEM((C, CHUNK), jnp.int32),
            pltpu.VMEM((CHUNK, E), jnp.float32),
            pltpu.VMEM((CHUNK, E), jnp.float32),
            pltpu.SemaphoreType.DMA,
        ],
    )
    def k(idx_h, z_h, out_h, iv, buf0, buf1, sem):
        wid = lax.axis_index("s") * nc + lax.axis_index("c")
        base = wid * (C * CHUNK)
        pltpu.sync_copy(idx_h.at[wid], iv)
        bufs = (buf0, buf1)

        def fire(c):
            return pltpu.async_copy(z_h.at[iv.at[c]], bufs[c % 2], sem)

        h = fire(0)
        for c in range(C):
            h.wait()
            if c + 1 < C:
                h = fire(c + 1)
            pltpu.sync_copy(bufs[c % 2], out_h.at[pl.ds(base + c * CHUNK,
                                                        CHUNK)])

    return k(idx2d, z)[0]


def _tc_unpack(pa, po):
    R = 2048

    def body(a_ref, o_ref, oa_ref, oo_ref):
        ua = lax.bitcast_convert_type(a_ref[...], jnp.uint32)
        uo = lax.bitcast_convert_type(o_ref[...], jnp.uint32)
        za = lax.bitcast_convert_type((ua & 0xFFFF).astype(jnp.uint16),
                                      jnp.bfloat16)
        zo = lax.bitcast_convert_type((uo >> 16).astype(jnp.uint16),
                                      jnp.bfloat16)
        oa_ref[...] = za.astype(jnp.float32)
        oo_ref[...] = zo.astype(jnp.float32)

    return pl.pallas_call(
        body,
        grid=(B // R,),
        in_specs=[
            pl.BlockSpec((R, E), lambda i: (i, 0)),
            pl.BlockSpec((R, E), lambda i: (i, 0)),
        ],
        out_specs=[
            pl.BlockSpec((R, E), lambda i: (i, 0)),
            pl.BlockSpec((R, E), lambda i: (i, 0)),
        ],
        out_shape=[
            jax.ShapeDtypeStruct((B, E), jnp.float32),
            jax.ShapeDtypeStruct((B, E), jnp.float32),
        ],
    )(pa, po)


def kernel(attrs, objs, attr_table, obj_table, W_attr, b_attr, W_obj, b_obj):
    info = plsc.get_sparse_core_info()
    nc, ns = info.num_cores, info.num_subcores
    NW = nc * ns
    C = B // NW // CHUNK
    a2 = attrs.astype(jnp.int32).reshape(NW, C, CHUNK)
    o2 = objs.astype(jnp.int32).reshape(NW, C, CHUNK)

    zpack = _tc_preproject_pack(attr_table.T, obj_table.T, W_attr, b_attr,
                                W_obj, b_obj)
    pa = _sc_gather(a2, zpack, nc, ns)
    po = _sc_gather(o2, zpack, nc, ns)
    oa, oo = _tc_unpack(pa, po)
    return oa, oo


# speedup vs baseline: 2.5777x; 1.0425x over previous
import functools

import jax
import jax.numpy as jnp
from jax import lax
from jax.experimental import pallas as pl
from jax.experimental.pallas import tpu as pltpu
from jax.experimental.pallas import tpu_sc as plsc

B = 16384
V = 100000
D = 300
E = 128
CHUNK = 128
NC = 2048


def _tc_preproject_pack(tabTa, tabTo, Wa, ba, Wo, bo):
    grid = (pl.cdiv(V, NC),)

    def one(x, w_ref, b_ref):
        x = jnp.where(x >= 0, x, 0.01 * x).astype(jnp.bfloat16)
        y = lax.dot_general(x, w_ref[...].astype(jnp.bfloat16),
                            (((0,), (0,)), ((), ())),
                            preferred_element_type=jnp.float32)
        y = y + b_ref[...][None, :]
        n = jnp.sqrt(jnp.sum(y * y, axis=1, keepdims=True))
        return y / jnp.maximum(n, 1e-12)

    def body(xa_ref, xo_ref, wa_ref, ba_ref, wo_ref, bo_ref, z_ref):
        za = one(xa_ref[...], wa_ref, ba_ref).astype(jnp.bfloat16)
        zo = one(xo_ref[...], wo_ref, bo_ref).astype(jnp.bfloat16)
        ua = lax.bitcast_convert_type(za, jnp.uint16).astype(jnp.uint32)
        uo = lax.bitcast_convert_type(zo, jnp.uint16).astype(jnp.uint32)
        packed = ua | (uo << 16)
        z_ref[...] = lax.bitcast_convert_type(packed, jnp.float32)

    return pl.pallas_call(
        body,
        grid=grid,
        in_specs=[
            pl.BlockSpec((D, NC), lambda i: (0, i)),
            pl.BlockSpec((D, NC), lambda i: (0, i)),
            pl.BlockSpec((D, E), lambda i: (0, 0)),
            pl.BlockSpec((E,), lambda i: (0,)),
            pl.BlockSpec((D, E), lambda i: (0, 0)),
            pl.BlockSpec((E,), lambda i: (0,)),
        ],
        out_specs=pl.BlockSpec((NC, E), lambda i: (i, 0)),
        out_shape=jax.ShapeDtypeStruct((V, E), jnp.float32),
    )(tabTa, tabTo, Wa, ba, Wo, bo)


def _sc_gather(idx2d, z, nc, ns):
    NW = nc * ns
    C = B // NW // CHUNK
    mesh = plsc.VectorSubcoreMesh(core_axis_name="c", subcore_axis_name="s")

    @functools.partial(
        pl.kernel,
        out_type=(jax.ShapeDtypeStruct((B, E), jnp.float32),),
        mesh=mesh,
        scratch_types=[
            pltpu.VMEM((C, CHUNK), jnp.int32),
            pltpu.VMEM((CHUNK, E), jnp.float32),
            pltpu.VMEM((CHUNK, E), jnp.float32),
            pltpu.SemaphoreType.DMA,
        ],
    )
    def k(idx_h, z_h, out_h, iv, buf0, buf1, sem):
        wid = lax.axis_index("s") * nc + lax.axis_index("c")
        base = wid * (C * CHUNK)
        pltpu.sync_copy(idx_h.at[wid], iv)
        bufs = (buf0, buf1)

        def fire(c):
            return pltpu.async_copy(z_h.at[iv.at[c]], bufs[c % 2], sem)

        h = fire(0)
        for c in range(C):
            h.wait()
            if c + 1 < C:
                h = fire(c + 1)
            pltpu.sync_copy(bufs[c % 2], out_h.at[pl.ds(base + c * CHUNK,
                                                        CHUNK)])

    return k(idx2d, z)[0]


def _tc_unpack(pa, po):
    R = 2048

    def body(a_ref, o_ref, oa_ref, oo_ref):
        ua = lax.bitcast_convert_type(a_ref[...], jnp.uint32)
        uo = lax.bitcast_convert_type(o_ref[...], jnp.uint32)
        za = lax.bitcast_convert_type((ua & 0xFFFF).astype(jnp.uint16),
                                      jnp.bfloat16)
        zo = lax.bitcast_convert_type((uo >> 16).astype(jnp.uint16),
                                      jnp.bfloat16)
        oa_ref[...] = za.astype(jnp.float32)
        oo_ref[...] = zo.astype(jnp.float32)

    return pl.pallas_call(
        body,
        grid=(B // R,),
        in_specs=[
            pl.BlockSpec((R, E), lambda i: (i, 0)),
            pl.BlockSpec((R, E), lambda i: (i, 0)),
        ],
        out_specs=[
            pl.BlockSpec((R, E), lambda i: (i, 0)),
            pl.BlockSpec((R, E), lambda i: (i, 0)),
        ],
        out_shape=[
            jax.ShapeDtypeStruct((B, E), jnp.float32),
            jax.ShapeDtypeStruct((B, E), jnp.float32),
        ],
    )(pa, po)


def kernel(attrs, objs, attr_table, obj_table, W_attr, b_attr, W_obj, b_obj):
    info = plsc.get_sparse_core_info()
    nc, ns = info.num_cores, info.num_subcores
    NW = nc * ns
    C = B // NW // CHUNK
    a2 = attrs.astype(jnp.int32).reshape(NW, C, CHUNK)
    o2 = objs.astype(jnp.int32).reshape(NW, C, CHUNK)

    zpack = _tc_preproject_pack(attr_table.T, obj_table.T, W_attr, b_attr,
                                W_obj, b_obj)
    pa = _sc_gather(a2, zpack, nc, ns)
    po = _sc_gather(o2, zpack, nc, ns)
    oa, oo = _tc_unpack(pa, po)
    return oa, oo


# speedup vs baseline: 2.6455x; 1.0263x over previous
import functools

import jax
import jax.numpy as jnp
from jax import lax
from jax.experimental import pallas as pl
from jax.experimental.pallas import tpu as pltpu
from jax.experimental.pallas import tpu_sc as plsc

B = 16384
V = 100000
D = 300
E = 128
CHUNK = 128
NC = 2048


def _tc_preproject_pack(tabTa, tabTo, Wa, ba, Wo, bo):
    grid = (pl.cdiv(V, NC),)

    def one(x, w_ref, b_ref):
        x = jnp.where(x >= 0, x, 0.01 * x).astype(jnp.bfloat16)
        y = lax.dot_general(x, w_ref[...].astype(jnp.bfloat16),
                            (((0,), (0,)), ((), ())),
                            preferred_element_type=jnp.float32)
        y = y + b_ref[...][None, :]
        n = jnp.sqrt(jnp.sum(y * y, axis=1, keepdims=True))
        return y / jnp.maximum(n, 1e-12)

    def body(xa_ref, xo_ref, wa_ref, ba_ref, wo_ref, bo_ref, z_ref):
        za = one(xa_ref[...], wa_ref, ba_ref).astype(jnp.bfloat16)
        zo = one(xo_ref[...], wo_ref, bo_ref).astype(jnp.bfloat16)
        ua = lax.bitcast_convert_type(za, jnp.uint16).astype(jnp.uint32)
        uo = lax.bitcast_convert_type(zo, jnp.uint16).astype(jnp.uint32)
        packed = ua | (uo << 16)
        z_ref[...] = lax.bitcast_convert_type(packed, jnp.float32)

    return pl.pallas_call(
        body,
        grid=grid,
        in_specs=[
            pl.BlockSpec((D, NC), lambda i: (0, i)),
            pl.BlockSpec((D, NC), lambda i: (0, i)),
            pl.BlockSpec((D, E), lambda i: (0, 0)),
            pl.BlockSpec((E,), lambda i: (0,)),
            pl.BlockSpec((D, E), lambda i: (0, 0)),
            pl.BlockSpec((E,), lambda i: (0,)),
        ],
        out_specs=pl.BlockSpec((NC, E), lambda i: (i, 0)),
        out_shape=jax.ShapeDtypeStruct((V, E), jnp.float32),
    )(tabTa, tabTo, Wa, ba, Wo, bo)


def _sc_gather_both(a2, o2, z, nc, ns):
    NW = nc * ns
    C = B // NW // CHUNK
    mesh = plsc.VectorSubcoreMesh(core_axis_name="c", subcore_axis_name="s")

    @functools.partial(
        pl.kernel,
        out_type=(
            jax.ShapeDtypeStruct((B, E), jnp.float32),
            jax.ShapeDtypeStruct((B, E), jnp.float32),
        ),
        mesh=mesh,
        scratch_types=[
            pltpu.VMEM((C, CHUNK), jnp.int32),
            pltpu.VMEM((C, CHUNK), jnp.int32),
            pltpu.VMEM((CHUNK, E), jnp.float32),
            pltpu.VMEM((CHUNK, E), jnp.float32),
            pltpu.SemaphoreType.DMA,
        ],
    )
    def k(aidx_h, oidx_h, z_h, pa_h, po_h, ia, io, buf0, buf1, sem):
        wid = lax.axis_index("s") * nc + lax.axis_index("c")
        base = wid * (C * CHUNK)
        pltpu.sync_copy(aidx_h.at[wid], ia)
        pltpu.sync_copy(oidx_h.at[wid], io)
        bufs = (buf0, buf1)
        jobs = [(ia, c, pa_h) for c in range(C)]
        jobs += [(io, c, po_h) for c in range(C)]

        def fire(j):
            idxr, c, _ = jobs[j]
            return pltpu.async_copy(z_h.at[idxr.at[c]], bufs[j % 2], sem)

        h = fire(0)
        for j in range(len(jobs)):
            h.wait()
            if j + 1 < len(jobs):
                h = fire(j + 1)
            _, c, out = jobs[j]
            pltpu.sync_copy(bufs[j % 2], out.at[pl.ds(base + c * CHUNK,
                                                      CHUNK)])

    return k(a2, o2, z)


def _tc_unpack(pa, po):
    R = 2048

    def body(a_ref, o_ref, oa_ref, oo_ref):
        ua = lax.bitcast_convert_type(a_ref[...], jnp.uint32)
        uo = lax.bitcast_convert_type(o_ref[...], jnp.uint32)
        za = lax.bitcast_convert_type((ua & 0xFFFF).astype(jnp.uint16),
                                      jnp.bfloat16)
        zo = lax.bitcast_convert_type((uo >> 16).astype(jnp.uint16),
                                      jnp.bfloat16)
        oa_ref[...] = za.astype(jnp.float32)
        oo_ref[...] = zo.astype(jnp.float32)

    return pl.pallas_call(
        body,
        grid=(B // R,),
        in_specs=[
            pl.BlockSpec((R, E), lambda i: (i, 0)),
            pl.BlockSpec((R, E), lambda i: (i, 0)),
        ],
        out_specs=[
            pl.BlockSpec((R, E), lambda i: (i, 0)),
            pl.BlockSpec((R, E), lambda i: (i, 0)),
        ],
        out_shape=[
            jax.ShapeDtypeStruct((B, E), jnp.float32),
            jax.ShapeDtypeStruct((B, E), jnp.float32),
        ],
    )(pa, po)


def kernel(attrs, objs, attr_table, obj_table, W_attr, b_attr, W_obj, b_obj):
    info = plsc.get_sparse_core_info()
    nc, ns = info.num_cores, info.num_subcores
    NW = nc * ns
    C = B // NW // CHUNK
    a2 = attrs.astype(jnp.int32).reshape(NW, C, CHUNK)
    o2 = objs.astype(jnp.int32).reshape(NW, C, CHUNK)

    zpack = _tc_preproject_pack(attr_table.T, obj_table.T, W_attr, b_attr,
                                W_obj, b_obj)
    pa, po = _sc_gather_both(a2, o2, zpack, nc, ns)
    oa, oo = _tc_unpack(pa, po)
    return oa, oo


# speedup vs baseline: 2.8431x; 1.0747x over previous
import functools

import jax
import jax.numpy as jnp
from jax import lax
from jax.experimental import pallas as pl
from jax.experimental.pallas import tpu as pltpu
from jax.experimental.pallas import tpu_sc as plsc

B = 16384
V = 100000
D = 300
E = 128
CHUNK = 128
NC = 4096


def _tc_preproject_pack(tabTa, tabTo, Wa, ba, Wo, bo):
    grid = (pl.cdiv(V, NC),)

    def one(x, w_ref, b_ref):
        x = jnp.where(x >= 0, x, 0.01 * x).astype(jnp.bfloat16)
        y = lax.dot_general(x, w_ref[...].astype(jnp.bfloat16),
                            (((0,), (0,)), ((), ())),
                            preferred_element_type=jnp.float32)
        y = y + b_ref[...][None, :]
        n = jnp.sqrt(jnp.sum(y * y, axis=1, keepdims=True))
        return y / jnp.maximum(n, 1e-12)

    def body(xa_ref, xo_ref, wa_ref, ba_ref, wo_ref, bo_ref, z_ref):
        za = one(xa_ref[...], wa_ref, ba_ref).astype(jnp.bfloat16)
        zo = one(xo_ref[...], wo_ref, bo_ref).astype(jnp.bfloat16)
        ua = lax.bitcast_convert_type(za, jnp.uint16).astype(jnp.uint32)
        uo = lax.bitcast_convert_type(zo, jnp.uint16).astype(jnp.uint32)
        packed = ua | (uo << 16)
        z_ref[...] = lax.bitcast_convert_type(packed, jnp.float32)

    return pl.pallas_call(
        body,
        grid=grid,
        in_specs=[
            pl.BlockSpec((D, NC), lambda i: (0, i)),
            pl.BlockSpec((D, NC), lambda i: (0, i)),
            pl.BlockSpec((D, E), lambda i: (0, 0)),
            pl.BlockSpec((E,), lambda i: (0,)),
            pl.BlockSpec((D, E), lambda i: (0, 0)),
            pl.BlockSpec((E,), lambda i: (0,)),
        ],
        out_specs=pl.BlockSpec((NC, E), lambda i: (i, 0)),
        out_shape=jax.ShapeDtypeStruct((V, E), jnp.float32),
    )(tabTa, tabTo, Wa, ba, Wo, bo)


def _sc_gather_both(a2, o2, z, nc, ns):
    NW = nc * ns
    C = B // NW // CHUNK
    mesh = plsc.VectorSubcoreMesh(core_axis_name="c", subcore_axis_name="s")

    @functools.partial(
        pl.kernel,
        out_type=(
            jax.ShapeDtypeStruct((B, E), jnp.float32),
            jax.ShapeDtypeStruct((B, E), jnp.float32),
        ),
        mesh=mesh,
        scratch_types=[
            pltpu.VMEM((C, CHUNK), jnp.int32),
            pltpu.VMEM((C, CHUNK), jnp.int32),
            pltpu.VMEM((CHUNK, E), jnp.float32),
            pltpu.VMEM((CHUNK, E), jnp.float32),
            pltpu.SemaphoreType.DMA,
        ],
    )
    def k(aidx_h, oidx_h, z_h, pa_h, po_h, ia, io, buf0, buf1, sem):
        wid = lax.axis_index("s") * nc + lax.axis_index("c")
        base = wid * (C * CHUNK)
        pltpu.sync_copy(aidx_h.at[wid], ia)
        pltpu.sync_copy(oidx_h.at[wid], io)
        bufs = (buf0, buf1)
        jobs = [(ia, c, pa_h) for c in range(C)]
        jobs += [(io, c, po_h) for c in range(C)]

        def fire(j):
            idxr, c, _ = jobs[j]
            return pltpu.async_copy(z_h.at[idxr.at[c]], bufs[j % 2], sem)

        h = fire(0)
        for j in range(len(jobs)):
            h.wait()
            if j + 1 < len(jobs):
                h = fire(j + 1)
            _, c, out = jobs[j]
            pltpu.sync_copy(bufs[j % 2], out.at[pl.ds(base + c * CHUNK,
                                                      CHUNK)])

    return k(a2, o2, z)


def _tc_unpack(pa, po):
    R = 2048

    def body(a_ref, o_ref, oa_ref, oo_ref):
        ua = lax.bitcast_convert_type(a_ref[...], jnp.uint32)
        uo = lax.bitcast_convert_type(o_ref[...], jnp.uint32)
        za = lax.bitcast_convert_type((ua & 0xFFFF).astype(jnp.uint16),
                                      jnp.bfloat16)
        zo = lax.bitcast_convert_type((uo >> 16).astype(jnp.uint16),
                                      jnp.bfloat16)
        oa_ref[...] = za.astype(jnp.float32)
        oo_ref[...] = zo.astype(jnp.float32)

    return pl.pallas_call(
        body,
        grid=(B // R,),
        in_specs=[
            pl.BlockSpec((R, E), lambda i: (i, 0)),
            pl.BlockSpec((R, E), lambda i: (i, 0)),
        ],
        out_specs=[
            pl.BlockSpec((R, E), lambda i: (i, 0)),
            pl.BlockSpec((R, E), lambda i: (i, 0)),
        ],
        out_shape=[
            jax.ShapeDtypeStruct((B, E), jnp.float32),
            jax.ShapeDtypeStruct((B, E), jnp.float32),
        ],
    )(pa, po)


def kernel(attrs, objs, attr_table, obj_table, W_attr, b_attr, W_obj, b_obj):
    info = plsc.get_sparse_core_info()
    nc, ns = info.num_cores, info.num_subcores
    NW = nc * ns
    C = B // NW // CHUNK
    a2 = attrs.astype(jnp.int32).reshape(NW, C, CHUNK)
    o2 = objs.astype(jnp.int32).reshape(NW, C, CHUNK)

    zpack = _tc_preproject_pack(attr_table.T, obj_table.T, W_attr, b_attr,
                                W_obj, b_obj)
    pa, po = _sc_gather_both(a2, o2, zpack, nc, ns)
    oa, oo = _tc_unpack(pa, po)
    return oa, oo


# speedup vs baseline: 2.8798x; 1.0129x over previous
import functools

import jax
import jax.numpy as jnp
from jax import lax
from jax.experimental import pallas as pl
from jax.experimental.pallas import tpu as pltpu
from jax.experimental.pallas import tpu_sc as plsc

B = 16384
V = 100000
D = 300
E = 128
CHUNK = 128
NC = 8192


def _tc_preproject_pack(tabTa, tabTo, Wa, ba, Wo, bo):
    grid = (pl.cdiv(V, NC),)

    def one(x, w_ref, b_ref):
        x = jnp.where(x >= 0, x, 0.01 * x).astype(jnp.bfloat16)
        y = lax.dot_general(x, w_ref[...].astype(jnp.bfloat16),
                            (((0,), (0,)), ((), ())),
                            preferred_element_type=jnp.float32)
        y = y + b_ref[...][None, :]
        n = jnp.sqrt(jnp.sum(y * y, axis=1, keepdims=True))
        return y / jnp.maximum(n, 1e-12)

    def body(xa_ref, xo_ref, wa_ref, ba_ref, wo_ref, bo_ref, z_ref):
        za = one(xa_ref[...], wa_ref, ba_ref).astype(jnp.bfloat16)
        zo = one(xo_ref[...], wo_ref, bo_ref).astype(jnp.bfloat16)
        ua = lax.bitcast_convert_type(za, jnp.uint16).astype(jnp.uint32)
        uo = lax.bitcast_convert_type(zo, jnp.uint16).astype(jnp.uint32)
        packed = ua | (uo << 16)
        z_ref[...] = lax.bitcast_convert_type(packed, jnp.float32)

    return pl.pallas_call(
        body,
        grid=grid,
        in_specs=[
            pl.BlockSpec((D, NC), lambda i: (0, i)),
            pl.BlockSpec((D, NC), lambda i: (0, i)),
            pl.BlockSpec((D, E), lambda i: (0, 0)),
            pl.BlockSpec((E,), lambda i: (0,)),
            pl.BlockSpec((D, E), lambda i: (0, 0)),
            pl.BlockSpec((E,), lambda i: (0,)),
        ],
        out_specs=pl.BlockSpec((NC, E), lambda i: (i, 0)),
        out_shape=jax.ShapeDtypeStruct((V, E), jnp.float32),
    )(tabTa, tabTo, Wa, ba, Wo, bo)


def _sc_gather_both(a2, o2, z, nc, ns):
    NW = nc * ns
    C = B // NW // CHUNK
    mesh = plsc.VectorSubcoreMesh(core_axis_name="c", subcore_axis_name="s")

    @functools.partial(
        pl.kernel,
        out_type=(
            jax.ShapeDtypeStruct((B, E), jnp.float32),
            jax.ShapeDtypeStruct((B, E), jnp.float32),
        ),
        mesh=mesh,
        scratch_types=[
            pltpu.VMEM((C, CHUNK), jnp.int32),
            pltpu.VMEM((C, CHUNK), jnp.int32),
            pltpu.VMEM((CHUNK, E), jnp.float32),
            pltpu.VMEM((CHUNK, E), jnp.float32),
            pltpu.SemaphoreType.DMA,
        ],
    )
    def k(aidx_h, oidx_h, z_h, pa_h, po_h, ia, io, buf0, buf1, sem):
        wid = lax.axis_index("s") * nc + lax.axis_index("c")
        base = wid * (C * CHUNK)
        pltpu.sync_copy(aidx_h.at[wid], ia)
        pltpu.sync_copy(oidx_h.at[wid], io)
        bufs = (buf0, buf1)
        jobs = [(ia, c, pa_h) for c in range(C)]
        jobs += [(io, c, po_h) for c in range(C)]

        def fire(j):
            idxr, c, _ = jobs[j]
            return pltpu.async_copy(z_h.at[idxr.at[c]], bufs[j % 2], sem)

        h = fire(0)
        for j in range(len(jobs)):
            h.wait()
            if j + 1 < len(jobs):
                h = fire(j + 1)
            _, c, out = jobs[j]
            pltpu.sync_copy(bufs[j % 2], out.at[pl.ds(base + c * CHUNK,
                                                      CHUNK)])

    return k(a2, o2, z)


def _tc_unpack(pa, po):
    R = 2048

    def body(a_ref, o_ref, oa_ref, oo_ref):
        ua = lax.bitcast_convert_type(a_ref[...], jnp.uint32)
        uo = lax.bitcast_convert_type(o_ref[...], jnp.uint32)
        za = lax.bitcast_convert_type((ua & 0xFFFF).astype(jnp.uint16),
                                      jnp.bfloat16)
        zo = lax.bitcast_convert_type((uo >> 16).astype(jnp.uint16),
                                      jnp.bfloat16)
        oa_ref[...] = za.astype(jnp.float32)
        oo_ref[...] = zo.astype(jnp.float32)

    return pl.pallas_call(
        body,
        grid=(B // R,),
        in_specs=[
            pl.BlockSpec((R, E), lambda i: (i, 0)),
            pl.BlockSpec((R, E), lambda i: (i, 0)),
        ],
        out_specs=[
            pl.BlockSpec((R, E), lambda i: (i, 0)),
            pl.BlockSpec((R, E), lambda i: (i, 0)),
        ],
        out_shape=[
            jax.ShapeDtypeStruct((B, E), jnp.float32),
            jax.ShapeDtypeStruct((B, E), jnp.float32),
        ],
    )(pa, po)


def kernel(attrs, objs, attr_table, obj_table, W_attr, b_attr, W_obj, b_obj):
    info = plsc.get_sparse_core_info()
    nc, ns = info.num_cores, info.num_subcores
    NW = nc * ns
    C = B // NW // CHUNK
    a2 = attrs.astype(jnp.int32).reshape(NW, C, CHUNK)
    o2 = objs.astype(jnp.int32).reshape(NW, C, CHUNK)

    zpack = _tc_preproject_pack(attr_table.T, obj_table.T, W_attr, b_attr,
                                W_obj, b_obj)
    pa, po = _sc_gather_both(a2, o2, zpack, nc, ns)
    oa, oo = _tc_unpack(pa, po)
    return oa, oo
